# baseline (speedup 1.0000x reference)
import dataclasses
import functools

import jax
import jax.numpy as jnp
from jax import lax
from jax.experimental import pallas as pl
from jax.experimental.pallas import tpu as pltpu
from jax.experimental.pallas import tpu_sc as plsc

N = 100000
D = 256
N_SC = 8000
W = 80
NW = 32
L = 16

P = N_SC // 2
NCHUNK = P // W
KMAX = -(-NCHUNK // NW)

B = 4000
GRID = N // B
SC_BLOCKS = N_SC // B


def _sc_gather(table, xflat):
    mesh = plsc.VectorSubcoreMesh(core_axis_name="c", subcore_axis_name="s")
    cp = pltpu.CompilerParams()
    if "needs_layout_passes" in pltpu.CompilerParams.__dataclass_fields__:
        cp = dataclasses.replace(cp, needs_layout_passes=False)

    @functools.partial(
        pl.kernel,
        mesh=mesh,
        compiler_params=cp,
        out_type=jax.ShapeDtypeStruct((P, 2 * D), jnp.float32),
        scratch_types=[
            pltpu.VMEM((4 * W,), jnp.int32),
            pltpu.VMEM((W,), jnp.int32),
            pltpu.VMEM((W, 2 * D), jnp.float32),
            pltpu.VMEM((W, 2 * D), jnp.float32),
            pltpu.SemaphoreType.DMA,
            pltpu.SemaphoreType.DMA,
            pltpu.SemaphoreType.DMA,
        ],
    )
    def k(table_hbm, xflat_hbm, out_hbm, xv, idxv, rows0, rows1,
          gsem, wsem0, wsem1):
        wid = lax.axis_index("s") * 2 + lax.axis_index("c")
        rows = (rows0, rows1)
        wsem = (wsem0, wsem1)
        iota = lax.iota(jnp.int32, L)

        def fetch_gather_write(chunk, b):
            pltpu.sync_copy(xflat_hbm.at[pl.ds(chunk * 4 * W, 4 * W)], xv)
            for g in range(W // L):
                v = plsc.load_gather(xv, [iota * 4 + (4 * L * g)])
                for i in range(1, 4):
                    a = plsc.load_gather(xv, [iota * 4 + (4 * L * g + i)])
                    v = v * 3 + a
                idxv[pl.ds(g * L, L)] = v
            pltpu.async_copy(table_hbm.at[idxv], rows[b], gsem).wait()
            pltpu.async_copy(rows[b], out_hbm.at[pl.ds(chunk * W, W)],
                             wsem[b])

        def wait_write(b):
            pltpu.make_async_copy(
                rows[b], out_hbm.at[pl.ds(0, W)], wsem[b]).wait()

        fetch_gather_write(wid, 0)

        @pl.when(wid + NW < NCHUNK)
        def _():
            fetch_gather_write(wid + NW, 1)

        wait_write(0)

        @pl.when(wid + NW < NCHUNK)
        def _():
            wait_write(1)

    return k(table, xflat)


def _tc_combine(xt, thi, out_sc):
    def body(x_ref, thi_ref, sc_ref, o_ref):
        pid = pl.program_id(0)

        @pl.when(pid < SC_BLOCKS)
        def _():
            o_ref[...] = sc_ref[...]

        @pl.when(pid >= SC_BLOCKS)
        def _():
            i0 = x_ref[0, 0:1, :]
            i1 = x_ref[0, 1:2, :]
            row = lax.broadcasted_iota(jnp.int32, (8, B), 0)
            oht = ((row == i0) | (row == (i1 + 3))).astype(jnp.bfloat16)
            o_ref[...] = lax.dot_general(
                oht, thi_ref[...],
                dimension_numbers=(((0,), (0,)), ((), ())),
                preferred_element_type=jnp.float32)

    return pl.pallas_call(
        body,
        grid=(GRID,),
        in_specs=[
            pl.BlockSpec((1, 2, B), lambda i: (i, 0, 0)),
            pl.BlockSpec((8, D), lambda i: (0, 0)),
            pl.BlockSpec((B, D),
                         lambda i: (jnp.minimum(i, SC_BLOCKS - 1), 0)),
        ],
        out_specs=pl.BlockSpec((B, D), lambda i: (i, 0)),
        out_shape=jax.ShapeDtypeStruct((N, D), jnp.float32),
    )(xt, thi, out_sc)


def kernel(x, node_type_table, num_inv_pred_table):
    def cross(a, b):
        n, m = a.shape[0], b.shape[0]
        left = jnp.broadcast_to(a[:, None, :], (n, m, a.shape[1]))
        right = jnp.broadcast_to(b[None, :, :], (n, m, b.shape[1]))
        return jnp.concatenate([left, right], axis=2).reshape(
            n * m, a.shape[1] + b.shape[1])

    xi = x.astype(jnp.int32)

    c9 = cross(node_type_table, num_inv_pred_table)
    c81 = cross(c9, c9)
    xflat_sc = xi[:N_SC].reshape(-1)
    out_sc = _sc_gather(c81, xflat_sc)

    t = jnp.zeros((8, D), jnp.float32)
    t = t.at[0:3, :128].set(node_type_table)
    t = t.at[3:6, 128:].set(num_inv_pred_table)
    thi = t.astype(jnp.bfloat16)

    xt = xi.reshape(GRID, B, 2).transpose(0, 2, 1)
    return _tc_combine(xt, thi, out_sc.reshape(N_SC, D))
